# baseline (speedup 1.0000x reference)
import functools

import jax
import jax.numpy as jnp
from jax import lax
from jax.experimental import pallas as pl
from jax.experimental.pallas import tpu as pltpu
from jax.experimental.pallas import tpu_sc as plsc

D = 64
NC, NS = 2, 16
NW = NC * NS
CHUNK = 512
SUB = 128
LANES = 16

_DNUMS = lax.GatherDimensionNumbers(
    offset_dims=(), collapsed_slice_dims=(0,), start_index_map=(0,))


def _lane_broadcast(vec, k):
    idx = jnp.full((LANES, 1), k, dtype=jnp.int32)
    return lax.gather(vec, idx, _DNUMS, (1,),
                      mode=lax.GatherScatterMode.PROMISE_IN_BOUNDS)


def _sc_embed(e_idx, r_idx, c_idx, etab, rtab, ctab, n):
    n_per_w = n // NW
    n_chunks = n_per_w // CHUNK

    mesh = plsc.VectorSubcoreMesh(
        core_axis_name="c", subcore_axis_name="s",
        num_cores=NC, num_subcores=NS)

    @functools.partial(
        pl.kernel,
        out_type=jax.ShapeDtypeStruct((n, D), jnp.float32),
        mesh=mesh,
        scratch_types=[
            pltpu.VMEM((CHUNK,), jnp.int32),
            pltpu.VMEM((CHUNK,), jnp.int32),
            pltpu.VMEM((CHUNK,), jnp.int32),
            pltpu.VMEM((2, D), jnp.float32),
            pltpu.VMEM((CHUNK, D), jnp.float32),
            pltpu.VMEM((CHUNK, D), jnp.float32),
            pltpu.SemaphoreType.DMA,
        ],
        compiler_params=pltpu.CompilerParams(use_tc_tiling_on_sc=False),
    )
    def k(e_hbm, r_hbm, c_hbm, et_hbm, rt_hbm, ct_hbm, out_hbm,
          eiv, riv, civ, rtab_v, ebuf, cbuf, sem):
        wid = lax.axis_index("s") * NC + lax.axis_index("c")
        wbase = wid * n_per_w

        pltpu.sync_copy(rt_hbm, rtab_v)
        r0 = [rtab_v[0, pl.ds(d * LANES, LANES)] for d in range(D // LANES)]
        r1 = [rtab_v[1, pl.ds(d * LANES, LANES)] for d in range(D // LANES)]
        dlt = [a - b for a, b in zip(r1, r0)]

        def chunk_body(i, carry):
            base = wbase + i * CHUNK
            pltpu.sync_copy(e_hbm.at[pl.ds(base, CHUNK)], eiv)
            pltpu.sync_copy(r_hbm.at[pl.ds(base, CHUNK)], riv)
            copies = []
            for j in range(CHUNK // SUB):
                s = pl.ds(j * SUB, SUB)
                copies.append(pltpu.async_copy(et_hbm.at[eiv.at[s]], ebuf.at[s], sem))
            pltpu.sync_copy(c_hbm.at[pl.ds(base, CHUNK)], civ)
            for j in range(CHUNK // SUB):
                s = pl.ds(j * SUB, SUB)
                copies.append(pltpu.async_copy(ct_hbm.at[civ.at[s]], cbuf.at[s], sem))
            for cp in copies:
                cp.wait()

            def add_group(g, carry2):
                resp16 = riv[pl.ds(g * LANES, LANES)]
                for kk in range(LANES):
                    respf = _lane_broadcast(resp16, kk).astype(jnp.float32)
                    row = g * LANES + kk
                    for d in range(D // LANES):
                        sl = pl.ds(d * LANES, LANES)
                        ebuf[row, sl] = (ebuf[row, sl] + cbuf[row, sl]
                                         + r0[d] + respf * dlt[d])
                return carry2

            lax.fori_loop(0, CHUNK // LANES, add_group, 0)
            pltpu.sync_copy(ebuf, out_hbm.at[pl.ds(base, CHUNK)])
            return carry

        lax.fori_loop(0, n_chunks, chunk_body, 0)

    return k(e_idx, r_idx, c_idx, etab, rtab, ctab)


def kernel(exercises, response, concept, exercise_embed, response_embed, concept_embed):
    B, L = exercises.shape
    n = B * L
    e_idx = exercises.reshape(n).astype(jnp.int32)
    r_idx = response.reshape(n).astype(jnp.int32)
    c_idx = concept.reshape(n).astype(jnp.int32)
    out = _sc_embed(e_idx, r_idx, c_idx,
                    exercise_embed, response_embed, concept_embed, n)
    return out.reshape(B, L, D)


# speedup vs baseline: 1.0369x; 1.0369x over previous
import functools

import jax
import jax.numpy as jnp
from jax import lax
from jax.experimental import pallas as pl
from jax.experimental.pallas import tpu as pltpu
from jax.experimental.pallas import tpu_sc as plsc

D = 64
NC, NS = 2, 16
NW = NC * NS
CHUNK = 256
SUB = 128
LANES = 16
NBUF = 2

_DNUMS = lax.GatherDimensionNumbers(
    offset_dims=(), collapsed_slice_dims=(0,), start_index_map=(0,))


def _lane_broadcast(vec, k):
    idx = jnp.full((LANES, 1), k, dtype=jnp.int32)
    return lax.gather(vec, idx, _DNUMS, (1,),
                      mode=lax.GatherScatterMode.PROMISE_IN_BOUNDS)


def _sc_embed(e_idx, r_idx, c_idx, etab, rtab, ctab, n):
    n_per_w = n // NW
    n_chunks = n_per_w // CHUNK
    assert n_chunks % NBUF == 0

    mesh = plsc.VectorSubcoreMesh(
        core_axis_name="c", subcore_axis_name="s",
        num_cores=NC, num_subcores=NS)

    @functools.partial(
        pl.kernel,
        out_type=jax.ShapeDtypeStruct((n, D), jnp.float32),
        mesh=mesh,
        scratch_types=[
            pltpu.VMEM((NBUF, CHUNK), jnp.int32),
            pltpu.VMEM((NBUF, CHUNK), jnp.int32),
            pltpu.VMEM((NBUF, CHUNK), jnp.int32),
            pltpu.VMEM((2, D), jnp.float32),
            pltpu.VMEM((NBUF, CHUNK, D), jnp.float32),
            pltpu.VMEM((NBUF, CHUNK, D), jnp.float32),
            [pltpu.SemaphoreType.DMA] * NBUF,
        ],
        compiler_params=pltpu.CompilerParams(use_tc_tiling_on_sc=False),
    )
    def k(e_hbm, r_hbm, c_hbm, et_hbm, rt_hbm, ct_hbm, out_hbm,
          eiv, riv, civ, rtab_v, ebuf, cbuf, sems):
        wid = lax.axis_index("s") * NC + lax.axis_index("c")
        wbase = wid * n_per_w

        pltpu.sync_copy(rt_hbm, rtab_v)
        r0 = [rtab_v[0, pl.ds(d * LANES, LANES)] for d in range(D // LANES)]
        r1 = [rtab_v[1, pl.ds(d * LANES, LANES)] for d in range(D // LANES)]
        dlt = [a - b for a, b in zip(r1, r0)]

        def fire(i, b):
            base = wbase + i * CHUNK
            pltpu.sync_copy(e_hbm.at[pl.ds(base, CHUNK)], eiv.at[b])
            pltpu.sync_copy(r_hbm.at[pl.ds(base, CHUNK)], riv.at[b])
            pltpu.sync_copy(c_hbm.at[pl.ds(base, CHUNK)], civ.at[b])
            for j in range(CHUNK // SUB):
                s = pl.ds(j * SUB, SUB)
                pltpu.async_copy(et_hbm.at[eiv.at[b].at[s]],
                                 ebuf.at[b].at[s], sems[b])
                pltpu.async_copy(ct_hbm.at[civ.at[b].at[s]],
                                 cbuf.at[b].at[s], sems[b])

        def drain(b):
            for j in range(CHUNK // SUB):
                s = pl.ds(j * SUB, SUB)
                pltpu.make_async_copy(et_hbm.at[eiv.at[b].at[s]],
                                      ebuf.at[b].at[s], sems[b]).wait()
                pltpu.make_async_copy(ct_hbm.at[civ.at[b].at[s]],
                                      cbuf.at[b].at[s], sems[b]).wait()

        fire(0, 0)

        def pair_body(step, carry):
            for b in range(NBUF):
                i = NBUF * step + b
                nb = (b + 1) % NBUF

                @pl.when(i + 1 < n_chunks)
                def _():
                    fire(i + 1, nb)

                drain(b)

                def add_group(g, carry2):
                    resp16 = riv[b, pl.ds(g * LANES, LANES)]
                    for kk in range(LANES):
                        respf = _lane_broadcast(resp16, kk).astype(jnp.float32)
                        row = g * LANES + kk
                        for d in range(D // LANES):
                            sl = pl.ds(d * LANES, LANES)
                            ebuf[b, row, sl] = (ebuf[b, row, sl] + cbuf[b, row, sl]
                                                + r0[d] + respf * dlt[d])
                    return carry2

                lax.fori_loop(0, CHUNK // LANES, add_group, 0)
                pltpu.sync_copy(ebuf.at[b],
                                out_hbm.at[pl.ds(wbase + i * CHUNK, CHUNK)])
            return carry

        lax.fori_loop(0, n_chunks // NBUF, pair_body, 0)

    return k(e_idx, r_idx, c_idx, etab, rtab, ctab)


def kernel(exercises, response, concept, exercise_embed, response_embed, concept_embed):
    B, L = exercises.shape
    n = B * L
    e_idx = exercises.reshape(n).astype(jnp.int32)
    r_idx = response.reshape(n).astype(jnp.int32)
    c_idx = concept.reshape(n).astype(jnp.int32)
    out = _sc_embed(e_idx, r_idx, c_idx,
                    exercise_embed, response_embed, concept_embed, n)
    return out.reshape(B, L, D)
